# baseline (device time: 142625 ns/iter reference)
import jax
import jax.numpy as jnp
from jax import lax
from jax.experimental import pallas as pl
from jax.experimental.pallas import tpu as pltpu

N_DEV = 4
M_BLK = 1024
K_BLK = 1024
N_TOT = 8192
N_CHUNK = 512
N_CHUNKS = N_TOT // N_CHUNK
N_TILES = N_DEV * N_CHUNKS
N_SLOTS = 4


def _body(x_ref, w_ref, dummy_ref, out_ref, acc_ref, gath_ref, w_buf,
          send_sems, recv_sems, w_sems, out_sems):
    my = lax.axis_index("i")

    d_for_phase = [None, 1, 3, 2]
    k_order = [my] + [lax.rem(my + (N_DEV - d), N_DEV) for d in d_for_phase[1:]]

    def w_dma(t):
        p, c = divmod(t, N_CHUNKS)
        return pltpu.make_async_copy(
            w_ref.at[pl.ds(k_order[p] * K_BLK, K_BLK),
                     pl.ds(c * N_CHUNK, N_CHUNK)],
            w_buf.at[t % N_SLOTS],
            w_sems.at[t % N_SLOTS],
        )

    for t in range(N_SLOTS - 1):
        w_dma(t).start()

    barrier = pltpu.get_barrier_semaphore()
    for d in range(1, N_DEV):
        peer = lax.rem(my + d, N_DEV)
        pl.semaphore_signal(
            barrier, inc=1,
            device_id=(peer,), device_id_type=pl.DeviceIdType.MESH,
        )
    pl.semaphore_wait(barrier, N_DEV - 1)

    sends = []
    for d in range(1, N_DEV):
        t = lax.rem(my + d, N_DEV)
        rdma = pltpu.make_async_remote_copy(
            src_ref=x_ref.at[t],
            dst_ref=gath_ref.at[d - 1],
            send_sem=send_sems.at[d - 1],
            recv_sem=recv_sems.at[d - 1],
            device_id=(t,),
            device_id_type=pl.DeviceIdType.MESH,
        )
        rdma.start()
        sends.append(rdma)

    def out_dma(c):
        nsl = pl.ds(c * N_CHUNK, N_CHUNK)
        return pltpu.make_async_copy(
            acc_ref.at[:, nsl], out_ref.at[:, nsl],
            out_sems.at[c % N_SLOTS])

    lhs = x_ref[my].astype(jnp.float32)
    for t in range(N_TILES):
        p, c = divmod(t, N_CHUNKS)
        if c == 0 and p > 0:
            d = d_for_phase[p]
            recv = pltpu.make_async_remote_copy(
                src_ref=x_ref.at[k_order[p]],
                dst_ref=gath_ref.at[d - 1],
                send_sem=send_sems.at[d - 1],
                recv_sem=recv_sems.at[d - 1],
                device_id=(k_order[p],),
                device_id_type=pl.DeviceIdType.MESH,
            )
            recv.wait_recv()
            lhs = gath_ref[d - 1].astype(jnp.float32)
        w_dma(t).wait()
        nsl = pl.ds(c * N_CHUNK, N_CHUNK)
        part = lax.dot_general(
            lhs, w_buf[t % N_SLOTS],
            dimension_numbers=(((1,), (0,)), ((), ())),
            precision=lax.Precision.DEFAULT,
            preferred_element_type=jnp.float32,
        )
        if p == 0:
            acc_ref[:, nsl] = part
        elif p < N_DEV - 1:
            acc_ref[:, nsl] = acc_ref[:, nsl] + part
        else:
            acc_ref[:, nsl] = acc_ref[:, nsl] + part
            if c >= N_SLOTS:
                out_dma(c - N_SLOTS).wait()
            out_dma(c).start()
        if t + N_SLOTS - 1 < N_TILES:
            w_dma(t + N_SLOTS - 1).start()

    for c in range(N_CHUNKS - N_SLOTS, N_CHUNKS):
        out_dma(c).wait()
    for rdma in sends:
        rdma.wait_send()


def kernel(x, w_mat):
    x16 = x.astype(jnp.bfloat16).reshape(N_DEV, M_BLK, K_BLK)
    dummy = jnp.zeros((M_BLK, N_TOT), jnp.float32)
    return pl.pallas_call(
        _body,
        out_shape=jax.ShapeDtypeStruct((M_BLK, N_TOT), jnp.float32),
        in_specs=[
            pl.BlockSpec(memory_space=pltpu.VMEM),
            pl.BlockSpec(memory_space=pltpu.HBM),
            pl.BlockSpec(memory_space=pltpu.HBM),
        ],
        input_output_aliases={2: 0},
        out_specs=pl.BlockSpec(memory_space=pltpu.HBM),
        scratch_shapes=[
            pltpu.VMEM((M_BLK, N_TOT), jnp.float32),
            pltpu.VMEM((N_DEV - 1, M_BLK, K_BLK), jnp.bfloat16),
            pltpu.VMEM((N_SLOTS, K_BLK, N_CHUNK), jnp.float32),
            pltpu.SemaphoreType.DMA((N_DEV - 1,)),
            pltpu.SemaphoreType.DMA((N_DEV - 1,)),
            pltpu.SemaphoreType.DMA((N_SLOTS,)),
            pltpu.SemaphoreType.DMA((N_SLOTS,)),
        ],
        compiler_params=pltpu.CompilerParams(
            collective_id=0,
            vmem_limit_bytes=60 * 1024 * 1024,
        ),
    )(x16, w_mat, dummy)


# device time: 131105 ns/iter; 1.0879x vs baseline; 1.0879x over previous
import jax
import jax.numpy as jnp
from jax import lax
from jax.experimental import pallas as pl
from jax.experimental.pallas import tpu as pltpu

N_DEV = 4
M_BLK = 1024
K_BLK = 1024
N_TOT = 8192
N_CHUNK = 512
N_CHUNKS = N_TOT // N_CHUNK
N_TILES = N_DEV * N_CHUNKS
N_SLOTS = 4


def _body(x_ref, w_ref, out_ref, acc_ref, gath_ref, w_buf,
          send_sems, recv_sems, w_sems, out_sems):
    my = lax.axis_index("i")

    d_for_phase = [None, 1, 3, 2]
    k_order = [my] + [lax.rem(my + (N_DEV - d), N_DEV) for d in d_for_phase[1:]]

    def w_dma(t):
        p, c = divmod(t, N_CHUNKS)
        return pltpu.make_async_copy(
            w_ref.at[pl.ds(k_order[p] * K_BLK, K_BLK),
                     pl.ds(c * N_CHUNK, N_CHUNK)],
            w_buf.at[t % N_SLOTS],
            w_sems.at[t % N_SLOTS],
        )

    for t in range(N_SLOTS - 1):
        w_dma(t).start()

    barrier = pltpu.get_barrier_semaphore()
    for d in range(1, N_DEV):
        peer = lax.rem(my + d, N_DEV)
        pl.semaphore_signal(
            barrier, inc=1,
            device_id=(peer,), device_id_type=pl.DeviceIdType.MESH,
        )
    pl.semaphore_wait(barrier, N_DEV - 1)

    sends = []
    for d in range(1, N_DEV):
        t = lax.rem(my + d, N_DEV)
        rdma = pltpu.make_async_remote_copy(
            src_ref=x_ref.at[t],
            dst_ref=gath_ref.at[d - 1],
            send_sem=send_sems.at[d - 1],
            recv_sem=recv_sems.at[d - 1],
            device_id=(t,),
            device_id_type=pl.DeviceIdType.MESH,
        )
        rdma.start()
        sends.append(rdma)

    def out_dma(c):
        nsl = pl.ds(c * N_CHUNK, N_CHUNK)
        return pltpu.make_async_copy(
            acc_ref.at[:, nsl], out_ref.at[:, nsl],
            out_sems.at[c % N_SLOTS])

    lhs = x_ref[my].astype(jnp.float32)
    for t in range(N_TILES):
        p, c = divmod(t, N_CHUNKS)
        if c == 0 and p > 0:
            d = d_for_phase[p]
            recv = pltpu.make_async_remote_copy(
                src_ref=x_ref.at[k_order[p]],
                dst_ref=gath_ref.at[d - 1],
                send_sem=send_sems.at[d - 1],
                recv_sem=recv_sems.at[d - 1],
                device_id=(k_order[p],),
                device_id_type=pl.DeviceIdType.MESH,
            )
            recv.wait_recv()
            lhs = gath_ref[d - 1].astype(jnp.float32)
        w_dma(t).wait()
        nsl = pl.ds(c * N_CHUNK, N_CHUNK)
        part = lax.dot_general(
            lhs, w_buf[t % N_SLOTS],
            dimension_numbers=(((1,), (0,)), ((), ())),
            precision=lax.Precision.DEFAULT,
            preferred_element_type=jnp.float32,
        )
        if p == 0:
            acc_ref[:, nsl] = part
        elif p < N_DEV - 1:
            acc_ref[:, nsl] = acc_ref[:, nsl] + part
        else:
            acc_ref[:, nsl] = acc_ref[:, nsl] + part
            if c >= N_SLOTS:
                out_dma(c - N_SLOTS).wait()
            out_dma(c).start()
        if t + N_SLOTS - 1 < N_TILES:
            w_dma(t + N_SLOTS - 1).start()

    for c in range(N_CHUNKS - N_SLOTS, N_CHUNKS):
        out_dma(c).wait()
    for rdma in sends:
        rdma.wait_send()


def kernel(x, w_mat):
    x16 = x.astype(jnp.bfloat16).reshape(N_DEV, M_BLK, K_BLK)
    return pl.pallas_call(
        _body,
        out_shape=jax.ShapeDtypeStruct((M_BLK, N_TOT), jnp.float32),
        in_specs=[
            pl.BlockSpec(memory_space=pltpu.VMEM),
            pl.BlockSpec(memory_space=pltpu.HBM),
        ],
        out_specs=pl.BlockSpec(memory_space=pltpu.HBM),
        scratch_shapes=[
            pltpu.VMEM((M_BLK, N_TOT), jnp.float32),
            pltpu.VMEM((N_DEV - 1, M_BLK, K_BLK), jnp.bfloat16),
            pltpu.VMEM((N_SLOTS, K_BLK, N_CHUNK), jnp.float32),
            pltpu.SemaphoreType.DMA((N_DEV - 1,)),
            pltpu.SemaphoreType.DMA((N_DEV - 1,)),
            pltpu.SemaphoreType.DMA((N_SLOTS,)),
            pltpu.SemaphoreType.DMA((N_SLOTS,)),
        ],
        compiler_params=pltpu.CompilerParams(
            collective_id=0,
            vmem_limit_bytes=60 * 1024 * 1024,
        ),
    )(x16, w_mat)


# device time: 128519 ns/iter; 1.1098x vs baseline; 1.0201x over previous
import jax
import jax.numpy as jnp
from jax import lax
from jax.experimental import pallas as pl
from jax.experimental.pallas import tpu as pltpu

N_DEV = 4
M_BLK = 1024
K_BLK = 1024
N_TOT = 8192
N_CHUNK = 512
N_CHUNKS = N_TOT // N_CHUNK
N_TILES = N_DEV * N_CHUNKS
N_SLOTS = 4


def _body(x_ref, w_ref, out_ref, acc_ref, gath_ref, w_buf,
          send_sems, recv_sems, w_sems, out_sems):
    my = lax.axis_index("i")

    k_order = [my] + [lax.rem(my + (N_DEV - d), N_DEV) for d in (1, 3, 2)]

    tile_specs = (
        [(0, c) for c in range(N_CHUNKS)]
        + [(1, c) for c in range(N_CHUNKS)]
        + [pc for c in range(N_CHUNKS) for pc in ((2, c), (3, c))]
    )

    def w_dma(i):
        p, c = tile_specs[i]
        return pltpu.make_async_copy(
            w_ref.at[pl.ds(k_order[p] * K_BLK, K_BLK),
                     pl.ds(c * N_CHUNK, N_CHUNK)],
            w_buf.at[i % N_SLOTS],
            w_sems.at[i % N_SLOTS],
        )

    for i in range(N_SLOTS - 1):
        w_dma(i).start()
    own = pltpu.make_async_copy(
        x_ref.at[my], gath_ref.at[N_DEV - 1], out_sems.at[N_SLOTS])
    own.start()

    barrier = pltpu.get_barrier_semaphore()
    for d in range(1, N_DEV):
        peer = lax.rem(my + d, N_DEV)
        pl.semaphore_signal(
            barrier, inc=1,
            device_id=(peer,), device_id_type=pl.DeviceIdType.MESH,
        )
    pl.semaphore_wait(barrier, N_DEV - 1)

    sends = []
    for d in range(1, N_DEV):
        t = lax.rem(my + d, N_DEV)
        rdma = pltpu.make_async_remote_copy(
            src_ref=x_ref.at[t],
            dst_ref=gath_ref.at[d - 1],
            send_sem=send_sems.at[d - 1],
            recv_sem=recv_sems.at[d - 1],
            device_id=(t,),
            device_id_type=pl.DeviceIdType.MESH,
        )
        rdma.start()
        sends.append(rdma)

    def recv_wait(d):
        pltpu.make_async_remote_copy(
            src_ref=gath_ref.at[d - 1],
            dst_ref=gath_ref.at[d - 1],
            send_sem=send_sems.at[d - 1],
            recv_sem=recv_sems.at[d - 1],
            device_id=(my,),
            device_id_type=pl.DeviceIdType.MESH,
        ).wait_recv()

    def out_dma(c):
        nsl = pl.ds(c * N_CHUNK, N_CHUNK)
        return pltpu.make_async_copy(
            acc_ref.at[:, nsl], out_ref.at[:, nsl],
            out_sems.at[c % N_SLOTS])

    def dot(lhs, rhs):
        return lax.dot_general(
            lhs, rhs,
            dimension_numbers=(((1,), (0,)), ((), ())),
            precision=lax.Precision.DEFAULT,
            preferred_element_type=jnp.float32,
        )

    i = 0
    for p in range(2):
        if p == 0:
            own.wait()
            lhs = gath_ref[N_DEV - 1].astype(jnp.float32)
        else:
            recv_wait(1)
            lhs = gath_ref[0].astype(jnp.float32)
        for c in range(N_CHUNKS):
            w_dma(i).wait()
            nsl = pl.ds(c * N_CHUNK, N_CHUNK)
            part = dot(lhs, w_buf[i % N_SLOTS])
            if p == 0:
                acc_ref[:, nsl] = part
            else:
                acc_ref[:, nsl] = acc_ref[:, nsl] + part
            if i + N_SLOTS - 1 < N_TILES:
                w_dma(i + N_SLOTS - 1).start()
            i += 1

    recv_wait(3)
    lhs2 = gath_ref[2].astype(jnp.float32)
    recv_wait(2)
    lhs3 = gath_ref[1].astype(jnp.float32)
    for c in range(N_CHUNKS):
        w_dma(i).wait()
        pa = dot(lhs2, w_buf[i % N_SLOTS])
        if i + N_SLOTS - 1 < N_TILES:
            w_dma(i + N_SLOTS - 1).start()
        i += 1
        w_dma(i).wait()
        pb = dot(lhs3, w_buf[i % N_SLOTS])
        if i + N_SLOTS - 1 < N_TILES:
            w_dma(i + N_SLOTS - 1).start()
        i += 1
        nsl = pl.ds(c * N_CHUNK, N_CHUNK)
        acc_ref[:, nsl] = acc_ref[:, nsl] + (pa + pb)
        if c >= N_SLOTS:
            out_dma(c - N_SLOTS).wait()
        out_dma(c).start()

    for c in range(N_CHUNKS - N_SLOTS, N_CHUNKS):
        out_dma(c).wait()
    for rdma in sends:
        rdma.wait_send()


def kernel(x, w_mat):
    x16 = x.astype(jnp.bfloat16).reshape(N_DEV, M_BLK, K_BLK)
    return pl.pallas_call(
        _body,
        out_shape=jax.ShapeDtypeStruct((M_BLK, N_TOT), jnp.float32),
        in_specs=[
            pl.BlockSpec(memory_space=pltpu.HBM),
            pl.BlockSpec(memory_space=pltpu.HBM),
        ],
        out_specs=pl.BlockSpec(memory_space=pltpu.HBM),
        scratch_shapes=[
            pltpu.VMEM((M_BLK, N_TOT), jnp.float32),
            pltpu.VMEM((N_DEV, M_BLK, K_BLK), jnp.bfloat16),
            pltpu.VMEM((N_SLOTS, K_BLK, N_CHUNK), jnp.float32),
            pltpu.SemaphoreType.DMA((N_DEV - 1,)),
            pltpu.SemaphoreType.DMA((N_DEV - 1,)),
            pltpu.SemaphoreType.DMA((N_SLOTS,)),
            pltpu.SemaphoreType.DMA((N_SLOTS + 1,)),
        ],
        compiler_params=pltpu.CompilerParams(
            collective_id=0,
            vmem_limit_bytes=62 * 1024 * 1024,
        ),
    )(x16, w_mat)
